# baseline (speedup 1.0000x reference)
import functools

import jax
import jax.numpy as jnp
from jax import lax
from jax.experimental import pallas as pl
from jax.experimental.pallas import tpu as pltpu
from jax.experimental.pallas import tpu_sc as plsc

N = 10000
F_IN = 256
H = 512
C_OUT = 64
K = 10
ALPHA = 0.1

NCORE = 2
NSUB = 16
CH = 64
NCH = H // CH

N_PAD = 10240
RPT = N_PAD // NSUB
RH = RPT // 2

EBATCH = 128
NB = 42
NBUF = 7
EPT = NB * EBATCH
E_PAD = NCORE * NSUB * EPT
NROWS_E = E_PAD // EBATCH

ROWB = N_PAD // 8
TCG = 8

_mesh = plsc.VectorSubcoreMesh(core_axis_name="c", subcore_axis_name="s")
_sc_params = pltpu.CompilerParams(use_tc_tiling_on_sc=False)


@functools.partial(
    pl.kernel,
    mesh=_mesh,
    out_type=[jax.ShapeDtypeStruct((NCORE, N_PAD, CH), jnp.float32)
              for _ in range(NCH)],
    scratch_types=[
        pltpu.VMEM((NB, EBATCH), jnp.int32),
        pltpu.VMEM((NB, EBATCH), jnp.int32),
    ] + [pltpu.VMEM((EBATCH, CH), jnp.float32) for _ in range(NBUF)]
    + [
        pltpu.VMEM_SHARED((N_PAD, CH), jnp.float32),
        pltpu.SemaphoreType.DMA((NBUF,)),
        pltpu.SemaphoreType.DMA((NBUF,)),
    ],
    compiler_params=_sc_params,
)
def _sc_propagate(*refs):
    us = refs[:NCH]
    rows2, cols2, zeros_hbm = refs[NCH:NCH + 3]
    ags = refs[NCH + 3:NCH + 3 + NCH]
    row_all, col_all = refs[NCH + 3 + NCH:NCH + 5 + NCH]
    rbufs = refs[NCH + 5 + NCH:NCH + 5 + NCH + NBUF]
    acc, gsems, ssems = refs[-3:]
    c = lax.axis_index("c")
    s = lax.axis_index("s")
    base_bi = c * (NSUB * NB) + s * NB
    r0 = s * RPT
    pltpu.sync_copy(rows2.at[pl.ds(base_bi, NB)], row_all)
    pltpu.sync_copy(cols2.at[pl.ds(base_bi, NB)], col_all)
    for u, ag in zip(us, ags):
        def gather(b, j, u=u):
            return pltpu.make_async_copy(u.at[row_all.at[b]], rbufs[j],
                                         gsems.at[j])

        def scatter_start(b, j):
            pltpu.async_copy(rbufs[j], acc.at[col_all.at[b]], ssems.at[j],
                             add=True)

        def scatter_wait(b, j):
            pltpu.make_async_copy(rbufs[j], acc.at[col_all.at[b]],
                                  ssems.at[j]).wait()

        pltpu.sync_copy(zeros_hbm, acc.at[pl.ds(r0, RH)])
        pltpu.sync_copy(zeros_hbm, acc.at[pl.ds(r0 + RH, RH)])
        plsc.subcore_barrier()
        for j in range(NBUF):
            gather(j, j).start()

        @pl.loop(0, NB // NBUF)
        def _(i):
            b0 = NBUF * i
            for j in range(NBUF):
                gather(b0 + j, j).wait()
                scatter_start(b0 + j, j)
            for j in range(NBUF):
                @pl.when(b0 + j + NBUF < NB)
                def _(j=j, b0=b0):
                    scatter_wait(b0 + j, j)
                    gather(b0 + j + NBUF, j).start()

        for j in range(NBUF):
            scatter_wait(NB - NBUF + j, j)
        plsc.subcore_barrier()
        pltpu.sync_copy(acc.at[pl.ds(r0, RPT)], ag.at[c, pl.ds(r0, RPT)])
        plsc.subcore_barrier()


@functools.partial(
    pl.kernel,
    mesh=_mesh,
    out_type=[jax.ShapeDtypeStruct((NCORE, N_PAD, CH), jnp.float32)],
    scratch_types=[
        pltpu.VMEM((NB, EBATCH), jnp.int32),
        pltpu.VMEM((EBATCH, CH), jnp.float32),
        pltpu.VMEM((RH, CH), jnp.float32),
        pltpu.VMEM_SHARED((N_PAD, CH), jnp.float32),
    ],
    compiler_params=_sc_params,
)
def _sc_degree(cols2, ones_hbm, zeros_hbm, dag, col_all, ones_v, buf, acc):
    c = lax.axis_index("c")
    s = lax.axis_index("s")
    base_bi = c * (NSUB * NB) + s * NB
    r0 = s * RPT
    pltpu.sync_copy(cols2.at[pl.ds(base_bi, NB)], col_all)
    pltpu.sync_copy(ones_hbm, ones_v)
    pltpu.sync_copy(zeros_hbm, buf)
    pltpu.sync_copy(buf, acc.at[pl.ds(r0, RH)])
    pltpu.sync_copy(buf, acc.at[pl.ds(r0 + RH, RH)])
    plsc.subcore_barrier()

    @pl.loop(0, NB)
    def _(b):
        pltpu.sync_copy(ones_v, acc.at[col_all.at[b]], add=True)

    plsc.subcore_barrier()
    for half in range(2):
        pltpu.sync_copy(acc.at[pl.ds(r0 + half * RH, RH)], buf)
        pltpu.sync_copy(buf, dag.at[c, pl.ds(r0 + half * RH, RH)])
    plsc.subcore_barrier()


def _mm1_body(x_ref, w1_ref, b1_ref, dinv_ref, u_ref, v_ref):
    h0 = jnp.dot(x_ref[...], w1_ref[...], preferred_element_type=jnp.float32)
    h0 = jnp.maximum(h0 + b1_ref[...], 0.0)
    u0 = h0 * dinv_ref[...]
    u_ref[...] = u0
    v_ref[...] = ALPHA * u0


def _tc_mm1(x_pad, w1, b1, dinv_col):
    return pl.pallas_call(
        _mm1_body,
        grid=(TCG,),
        in_specs=[
            pl.BlockSpec((ROWB, F_IN), lambda i: (i, 0)),
            pl.BlockSpec((F_IN, H), lambda i: (0, 0)),
            pl.BlockSpec((1, H), lambda i: (0, 0)),
            pl.BlockSpec((ROWB, 1), lambda i: (i, 0)),
        ],
        out_specs=[
            pl.BlockSpec((ROWB, H), lambda i: (i, 0)),
            pl.BlockSpec((ROWB, H), lambda i: (i, 0)),
        ],
        out_shape=[jax.ShapeDtypeStruct((N_PAD, H), jnp.float32),
                   jax.ShapeDtypeStruct((N_PAD, H), jnp.float32)],
    )(x_pad, w1, b1, dinv_col)


def _fin_body(*refs):
    aggs = refs[:NCH]
    cb = refs[NCH][...]
    v0s = refs[NCH + 1:NCH + 1 + NCH]
    outs = refs[-NCH:]
    for a, v, o in zip(aggs, v0s, outs):
        o[...] = (a[0] + a[1]) * cb + v[...]


def _tc_finalize(aggs, c_col, v0ps):
    agg_spec = pl.BlockSpec((NCORE, ROWB, CH), lambda i: (0, i, 0))
    ch_spec = pl.BlockSpec((ROWB, CH), lambda i: (i, 0))
    return pl.pallas_call(
        _fin_body,
        grid=(TCG,),
        in_specs=[agg_spec] * NCH
        + [pl.BlockSpec((ROWB, 1), lambda i: (i, 0))]
        + [ch_spec] * NCH,
        out_specs=[ch_spec] * NCH,
        out_shape=[jax.ShapeDtypeStruct((N_PAD, CH), jnp.float32)
                   for _ in range(NCH)],
    )(*aggs, c_col, *v0ps)


def _mm2_body(*refs):
    uchunks = refs[:NCH]
    rdinv_ref, w2_ref, b2_ref, h_ref, lg_ref = refs[NCH:]
    u = jnp.concatenate([uc[...] for uc in uchunks], axis=1)
    h = jnp.maximum(u * rdinv_ref[...], 0.0)
    h_ref[...] = h
    lg = jnp.dot(h, w2_ref[...], preferred_element_type=jnp.float32)
    lg_ref[...] = lg + b2_ref[...]


def _tc_mm2(u_chunks, rdinv_col, w2, b2):
    ch_spec = pl.BlockSpec((ROWB, CH), lambda i: (i, 0))
    return pl.pallas_call(
        _mm2_body,
        grid=(TCG,),
        in_specs=[ch_spec] * NCH + [
            pl.BlockSpec((ROWB, 1), lambda i: (i, 0)),
            pl.BlockSpec((H, C_OUT), lambda i: (0, 0)),
            pl.BlockSpec((1, C_OUT), lambda i: (0, 0)),
        ],
        out_specs=[
            pl.BlockSpec((ROWB, H), lambda i: (i, 0)),
            pl.BlockSpec((ROWB, C_OUT), lambda i: (i, 0)),
        ],
        out_shape=[jax.ShapeDtypeStruct((N_PAD, H), jnp.float32),
                   jax.ShapeDtypeStruct((N_PAD, C_OUT), jnp.float32)],
    )(*u_chunks, rdinv_col, w2, b2)


def kernel(x, adj_t, dropout, W1, b1, W2, b2):
    del dropout
    e = adj_t.shape[1]
    pad_e = E_PAD - e - N
    loop = jnp.arange(N, dtype=jnp.int32)
    rows_full = jnp.concatenate(
        [adj_t[0].astype(jnp.int32), loop,
         jnp.zeros((pad_e,), jnp.int32)])
    cols_full = jnp.concatenate(
        [adj_t[1].astype(jnp.int32), loop,
         jnp.full((pad_e,), N, jnp.int32)])
    rows2 = rows_full.reshape(NROWS_E, EBATCH)
    cols2 = cols_full.reshape(NROWS_E, EBATCH)

    zeros_hbm = jnp.zeros((RH, CH), jnp.float32)
    ones_hbm = jnp.ones((EBATCH, CH), jnp.float32)

    (dag,) = _sc_degree(cols2, ones_hbm, zeros_hbm)
    deg = dag[0, :, 0] + dag[1, :, 0]
    valid = jnp.arange(N_PAD) < N
    deg_safe = jnp.maximum(deg, 1.0)
    dinv = jnp.where(valid, lax.rsqrt(deg_safe), 0.0)
    cvec = jnp.where(valid, (1.0 - ALPHA) / deg_safe, 0.0)
    rdinv = jnp.where(valid, jnp.sqrt(deg_safe), 0.0)

    x_pad = jnp.pad(x.astype(jnp.float32), ((0, N_PAD - N), (0, 0)))
    u_full, v0p_full = _tc_mm1(x_pad, W1.astype(jnp.float32),
                               b1.reshape(1, H).astype(jnp.float32),
                               dinv[:, None])
    u_chunks = [u_full[:, i * CH:(i + 1) * CH] for i in range(NCH)]
    v0p_chunks = [v0p_full[:, i * CH:(i + 1) * CH] for i in range(NCH)]

    c_col = cvec[:, None]
    for _ in range(K):
        aggs = _sc_propagate(*u_chunks, rows2, cols2, zeros_hbm)
        u_chunks = _tc_finalize(aggs, c_col, v0p_chunks)

    h_pad, logit_pad = _tc_mm2(u_chunks, rdinv[:, None],
                               W2.astype(jnp.float32),
                               b2.reshape(1, C_OUT).astype(jnp.float32))
    return (logit_pad[:N], h_pad[:N])


# speedup vs baseline: 1.0587x; 1.0587x over previous
import functools

import jax
import jax.numpy as jnp
from jax import lax
from jax.experimental import pallas as pl
from jax.experimental.pallas import tpu as pltpu
from jax.experimental.pallas import tpu_sc as plsc

N = 10000
F_IN = 256
H = 512
C_OUT = 64
K = 10
ALPHA = 0.1

NCORE = 2
NSUB = 16
CH = 64
NCH = H // CH

N_PAD = 10240
RPT = N_PAD // NSUB
RH = RPT // 2

EBATCH = 256
NB = 21
NBUF = 3
EPT = NB * EBATCH
E_PAD = NCORE * NSUB * EPT
NROWS_E = E_PAD // EBATCH

ROWB = N_PAD // 8
TCG = 8

_mesh = plsc.VectorSubcoreMesh(core_axis_name="c", subcore_axis_name="s")
_sc_params = pltpu.CompilerParams(use_tc_tiling_on_sc=False)


def _make_sc_propagate(nch):
  @functools.partial(
      pl.kernel,
      mesh=_mesh,
      out_type=[jax.ShapeDtypeStruct((NCORE, N_PAD, CH), jnp.float32)
                for _ in range(nch)],
      scratch_types=[
          pltpu.VMEM((NB, EBATCH), jnp.int32),
          pltpu.VMEM((NB, EBATCH), jnp.int32),
      ] + [pltpu.VMEM((EBATCH, CH), jnp.float32) for _ in range(NBUF)]
      + [
          pltpu.VMEM_SHARED((N_PAD, CH), jnp.float32),
          pltpu.SemaphoreType.DMA((NBUF,)),
          pltpu.SemaphoreType.DMA((NBUF,)),
      ],
      compiler_params=_sc_params,
  )
  def _sc_propagate(*refs):
    us = refs[:nch]
    rows2, cols2, zeros_hbm = refs[nch:nch + 3]
    ags = refs[nch + 3:nch + 3 + nch]
    row_all, col_all = refs[nch + 3 + nch:nch + 5 + nch]
    rbufs = refs[nch + 5 + nch:nch + 5 + nch + NBUF]
    acc, gsems, ssems = refs[-3:]
    c = lax.axis_index("c")
    s = lax.axis_index("s")
    base_bi = c * (NSUB * NB) + s * NB
    r0 = s * RPT
    pltpu.sync_copy(rows2.at[pl.ds(base_bi, NB)], row_all)
    pltpu.sync_copy(cols2.at[pl.ds(base_bi, NB)], col_all)
    for u, ag in zip(us, ags):
        def gather(b, j, u=u):
            return pltpu.make_async_copy(u.at[row_all.at[b]], rbufs[j],
                                         gsems.at[j])

        def scatter_start(b, j):
            pltpu.async_copy(rbufs[j], acc.at[col_all.at[b]], ssems.at[j],
                             add=True)

        def scatter_wait(b, j):
            pltpu.make_async_copy(rbufs[j], acc.at[col_all.at[b]],
                                  ssems.at[j]).wait()

        pltpu.sync_copy(zeros_hbm, acc.at[pl.ds(r0, RH)])
        pltpu.sync_copy(zeros_hbm, acc.at[pl.ds(r0 + RH, RH)])
        plsc.subcore_barrier()
        for j in range(NBUF):
            gather(j, j).start()

        @pl.loop(0, NB // NBUF)
        def _(i):
            b0 = NBUF * i
            for j in range(NBUF):
                gather(b0 + j, j).wait()
                scatter_start(b0 + j, j)
            for j in range(NBUF):
                @pl.when(b0 + j + NBUF < NB)
                def _(j=j, b0=b0):
                    scatter_wait(b0 + j, j)
                    gather(b0 + j + NBUF, j).start()

        for j in range(NBUF):
            scatter_wait(NB - NBUF + j, j)
        plsc.subcore_barrier()
        pltpu.sync_copy(acc.at[pl.ds(r0, RPT)], ag.at[c, pl.ds(r0, RPT)])
        plsc.subcore_barrier()

  return _sc_propagate


NCH_HALF = NCH // 2
_sc_propagate_half = _make_sc_propagate(NCH_HALF)


@functools.partial(
    pl.kernel,
    mesh=_mesh,
    out_type=[jax.ShapeDtypeStruct((NCORE, N_PAD, CH), jnp.float32)],
    scratch_types=[
        pltpu.VMEM((NB, EBATCH), jnp.int32),
        pltpu.VMEM((EBATCH, CH), jnp.float32),
        pltpu.VMEM((RH, CH), jnp.float32),
        pltpu.VMEM_SHARED((N_PAD, CH), jnp.float32),
    ],
    compiler_params=_sc_params,
)
def _sc_degree(cols2, ones_hbm, zeros_hbm, dag, col_all, ones_v, buf, acc):
    c = lax.axis_index("c")
    s = lax.axis_index("s")
    base_bi = c * (NSUB * NB) + s * NB
    r0 = s * RPT
    pltpu.sync_copy(cols2.at[pl.ds(base_bi, NB)], col_all)
    pltpu.sync_copy(ones_hbm, ones_v)
    pltpu.sync_copy(zeros_hbm, buf)
    pltpu.sync_copy(buf, acc.at[pl.ds(r0, RH)])
    pltpu.sync_copy(buf, acc.at[pl.ds(r0 + RH, RH)])
    plsc.subcore_barrier()

    @pl.loop(0, NB)
    def _(b):
        pltpu.sync_copy(ones_v, acc.at[col_all.at[b]], add=True)

    plsc.subcore_barrier()
    for half in range(2):
        pltpu.sync_copy(acc.at[pl.ds(r0 + half * RH, RH)], buf)
        pltpu.sync_copy(buf, dag.at[c, pl.ds(r0 + half * RH, RH)])
    plsc.subcore_barrier()


def _mm1_body(x_ref, w1_ref, b1_ref, dinv_ref, u_ref, v_ref):
    h0 = jnp.dot(x_ref[...], w1_ref[...], preferred_element_type=jnp.float32)
    h0 = jnp.maximum(h0 + b1_ref[...], 0.0)
    u0 = h0 * dinv_ref[...]
    u_ref[...] = u0
    v_ref[...] = ALPHA * u0


def _tc_mm1(x_pad, w1, b1, dinv_col):
    return pl.pallas_call(
        _mm1_body,
        grid=(TCG,),
        in_specs=[
            pl.BlockSpec((ROWB, F_IN), lambda i: (i, 0)),
            pl.BlockSpec((F_IN, H), lambda i: (0, 0)),
            pl.BlockSpec((1, H), lambda i: (0, 0)),
            pl.BlockSpec((ROWB, 1), lambda i: (i, 0)),
        ],
        out_specs=[
            pl.BlockSpec((ROWB, H), lambda i: (i, 0)),
            pl.BlockSpec((ROWB, H), lambda i: (i, 0)),
        ],
        out_shape=[jax.ShapeDtypeStruct((N_PAD, H), jnp.float32),
                   jax.ShapeDtypeStruct((N_PAD, H), jnp.float32)],
    )(x_pad, w1, b1, dinv_col)


def _fin_body(*refs):
    aggs = refs[:NCH_HALF]
    cb = refs[NCH_HALF][...]
    v0s = refs[NCH_HALF + 1:NCH_HALF + 1 + NCH_HALF]
    outs = refs[-NCH_HALF:]
    for a, v, o in zip(aggs, v0s, outs):
        o[...] = (a[0] + a[1]) * cb + v[...]


def _tc_finalize(aggs, c_col, v0ps):
    agg_spec = pl.BlockSpec((NCORE, ROWB, CH), lambda i: (0, i, 0))
    ch_spec = pl.BlockSpec((ROWB, CH), lambda i: (i, 0))
    return pl.pallas_call(
        _fin_body,
        grid=(TCG,),
        in_specs=[agg_spec] * NCH_HALF
        + [pl.BlockSpec((ROWB, 1), lambda i: (i, 0))]
        + [ch_spec] * NCH_HALF,
        out_specs=[ch_spec] * NCH_HALF,
        out_shape=[jax.ShapeDtypeStruct((N_PAD, CH), jnp.float32)
                   for _ in range(NCH_HALF)],
    )(*aggs, c_col, *v0ps)


def _mm2_body(*refs):
    uchunks = refs[:NCH]
    rdinv_ref, w2_ref, b2_ref, h_ref, lg_ref = refs[NCH:]
    u = jnp.concatenate([uc[...] for uc in uchunks], axis=1)
    h = jnp.maximum(u * rdinv_ref[...], 0.0)
    h_ref[...] = h
    lg = jnp.dot(h, w2_ref[...], preferred_element_type=jnp.float32)
    lg_ref[...] = lg + b2_ref[...]


def _tc_mm2(u_chunks, rdinv_col, w2, b2):
    ch_spec = pl.BlockSpec((ROWB, CH), lambda i: (i, 0))
    return pl.pallas_call(
        _mm2_body,
        grid=(TCG,),
        in_specs=[ch_spec] * NCH + [
            pl.BlockSpec((ROWB, 1), lambda i: (i, 0)),
            pl.BlockSpec((H, C_OUT), lambda i: (0, 0)),
            pl.BlockSpec((1, C_OUT), lambda i: (0, 0)),
        ],
        out_specs=[
            pl.BlockSpec((ROWB, H), lambda i: (i, 0)),
            pl.BlockSpec((ROWB, C_OUT), lambda i: (i, 0)),
        ],
        out_shape=[jax.ShapeDtypeStruct((N_PAD, H), jnp.float32),
                   jax.ShapeDtypeStruct((N_PAD, C_OUT), jnp.float32)],
    )(*u_chunks, rdinv_col, w2, b2)


def kernel(x, adj_t, dropout, W1, b1, W2, b2):
    del dropout
    e = adj_t.shape[1]
    pad_e = E_PAD - e - N
    loop = jnp.arange(N, dtype=jnp.int32)
    rows_full = jnp.concatenate(
        [adj_t[0].astype(jnp.int32), loop,
         jnp.zeros((pad_e,), jnp.int32)])
    cols_full = jnp.concatenate(
        [adj_t[1].astype(jnp.int32), loop,
         jnp.full((pad_e,), N, jnp.int32)])
    rows2 = rows_full.reshape(NROWS_E, EBATCH)
    cols2 = cols_full.reshape(NROWS_E, EBATCH)

    zeros_hbm = jnp.zeros((RH, CH), jnp.float32)
    ones_hbm = jnp.ones((EBATCH, CH), jnp.float32)

    (dag,) = _sc_degree(cols2, ones_hbm, zeros_hbm)
    deg = dag[0, :, 0] + dag[1, :, 0]
    valid = jnp.arange(N_PAD) < N
    deg_safe = jnp.maximum(deg, 1.0)
    dinv = jnp.where(valid, lax.rsqrt(deg_safe), 0.0)
    cvec = jnp.where(valid, (1.0 - ALPHA) / deg_safe, 0.0)
    rdinv = jnp.where(valid, jnp.sqrt(deg_safe), 0.0)

    x_pad = jnp.pad(x.astype(jnp.float32), ((0, N_PAD - N), (0, 0)))
    u_full, v0p_full = _tc_mm1(x_pad, W1.astype(jnp.float32),
                               b1.reshape(1, H).astype(jnp.float32),
                               dinv[:, None])
    u_chunks = [u_full[:, i * CH:(i + 1) * CH] for i in range(NCH)]
    v0p_chunks = [v0p_full[:, i * CH:(i + 1) * CH] for i in range(NCH)]

    c_col = cvec[:, None]
    nh = NCH_HALF
    for _ in range(K):
        aggs_a = _sc_propagate_half(*u_chunks[:nh], rows2, cols2, zeros_hbm)
        u_a = _tc_finalize(aggs_a, c_col, v0p_chunks[:nh])
        aggs_b = _sc_propagate_half(*u_chunks[nh:], rows2, cols2, zeros_hbm)
        u_b = _tc_finalize(aggs_b, c_col, v0p_chunks[nh:])
        u_chunks = list(u_a) + list(u_b)

    h_pad, logit_pad = _tc_mm2(u_chunks, rdinv[:, None],
                               W2.astype(jnp.float32),
                               b2.reshape(1, C_OUT).astype(jnp.float32))
    return (logit_pad[:N], h_pad[:N])
